# Initial kernel scaffold; baseline (speedup 1.0000x reference)
import jax
import jax.numpy as jnp
from jax.experimental import pallas as pl


def kernel(x, edge_index, batch, W1, b1, W2, b2, W3, b3, W4, b4):
    raise NotImplementedError("write your pallas kernel here")


# speedup vs baseline: 11.9797x; 11.9797x over previous
import functools

import jax
import jax.numpy as jnp
from jax import lax
from jax.experimental import pallas as pl
from jax.experimental.pallas import tpu as pltpu
from jax.experimental.pallas import tpu_sc as plsc

N = 10000
E = 320000
D = 128
G = 64
NC = 2
NS = 16
NW = NC * NS
C = 128
NCH = -(-E // (NW * C))
EPW = NCH * C
RPT = 632
NP = NS * RPT


def _sc_mesh():
    return plsc.VectorSubcoreMesh(
        core_axis_name="c", subcore_axis_name="s", num_cores=NC, num_subcores=NS
    )


@functools.partial(
    pl.kernel,
    out_type=jax.ShapeDtypeStruct((NC, NP, D), jnp.float32),
    mesh=_sc_mesh(),
    scratch_types=[
        pltpu.VMEM((NCH, C), jnp.int32),
        pltpu.VMEM((C, D), jnp.float32),
        pltpu.VMEM_SHARED((NP, D), jnp.float32),
    ],
)
def _deg_pass(dst_hbm, ones_hbm, z_hbm, out_hbm, idx_v, ones_v, acc):
    c = lax.axis_index("c")
    s = lax.axis_index("s")
    wid = c * NS + s
    pltpu.sync_copy(z_hbm, acc.at[pl.ds(s * RPT, RPT)])
    pltpu.sync_copy(dst_hbm.at[wid], idx_v)
    pltpu.sync_copy(ones_hbm, ones_v)
    plsc.subcore_barrier()

    def body(j, carry):
        pltpu.sync_copy(ones_v, acc.at[idx_v.at[j]], add=True)
        return carry

    lax.fori_loop(0, NCH, body, 0)
    plsc.subcore_barrier()
    pltpu.sync_copy(acc.at[pl.ds(s * RPT, RPT)],
                    out_hbm.at[c, pl.ds(s * RPT, RPT)])


@functools.partial(
    pl.kernel,
    out_type=jax.ShapeDtypeStruct((NC, NP, D), jnp.float32),
    mesh=_sc_mesh(),
    scratch_types=[
        pltpu.VMEM((NCH, C), jnp.int32),
        pltpu.VMEM((NCH, C), jnp.int32),
        pltpu.VMEM((C, D), jnp.float32),
        pltpu.VMEM_SHARED((NP, D), jnp.float32),
        pltpu.SemaphoreType.DMA,
    ],
)
def _edge_pass(g_hbm, src_hbm, dst_hbm, z_hbm, out_hbm,
               isrc_v, idst_v, rows_v, acc, sem):
    c = lax.axis_index("c")
    s = lax.axis_index("s")
    wid = c * NS + s
    pltpu.sync_copy(z_hbm, acc.at[pl.ds(s * RPT, RPT)])
    pltpu.sync_copy(src_hbm.at[wid], isrc_v)
    pltpu.sync_copy(dst_hbm.at[wid], idst_v)
    plsc.subcore_barrier()

    def body(j, carry):
        pltpu.async_copy(g_hbm.at[isrc_v.at[j]], rows_v, sem).wait()
        pltpu.sync_copy(rows_v, acc.at[idst_v.at[j]], add=True)
        return carry

    lax.fori_loop(0, NCH, body, 0)
    plsc.subcore_barrier()
    pltpu.sync_copy(acc.at[pl.ds(s * RPT, RPT)],
                    out_hbm.at[c, pl.ds(s * RPT, RPT)])


def _scale_body(degp_ref, x_ref, w_ref, g_ref, dinv_ref):
    deg = 1.0 + degp_ref[0, :, 0:1] + degp_ref[1, :, 0:1]
    dinv = lax.rsqrt(deg)
    ht = jnp.dot(x_ref[...], w_ref[...], preferred_element_type=jnp.float32)
    g_ref[...] = dinv * ht
    dinv_ref[...] = dinv


_scale_call = pl.pallas_call(
    _scale_body,
    out_shape=(
        jax.ShapeDtypeStruct((NP, D), jnp.float32),
        jax.ShapeDtypeStruct((NP, 1), jnp.float32),
    ),
)


def _layer_body(s_ref, g_ref, dinv_ref, b_ref, w_ref, out_ref):
    agg = s_ref[0] + s_ref[1] + g_ref[...]
    h = jnp.maximum(dinv_ref[...] * agg + b_ref[...], 0.0)
    out_ref[...] = dinv_ref[...] * jnp.dot(
        h, w_ref[...], preferred_element_type=jnp.float32)


_layer_call = pl.pallas_call(
    _layer_body,
    out_shape=jax.ShapeDtypeStruct((NP, D), jnp.float32),
)


def _head_body(s_ref, g_ref, dinv_ref, b2_ref, batch_ref,
               w3_ref, b3_ref, w4_ref, b4_ref, out_ref):
    agg = s_ref[0] + s_ref[1] + g_ref[...]
    h = jnp.maximum(dinv_ref[...] * agg + b2_ref[...], 0.0)
    hr = h[:N]
    onehot_t = (lax.broadcasted_iota(jnp.int32, (G, N), 0)
                == batch_ref[...]).astype(jnp.float32)
    psum = jnp.dot(onehot_t, hr, preferred_element_type=jnp.float32)
    cnt = jnp.sum(onehot_t, axis=1, keepdims=True)
    pooled = psum / jnp.maximum(cnt, 1.0)
    z = jnp.maximum(
        jnp.dot(pooled, w3_ref[...], preferred_element_type=jnp.float32)
        + b3_ref[...], 0.0)
    out_ref[...] = jnp.dot(
        z, w4_ref[...], preferred_element_type=jnp.float32) + b4_ref[...]


_head_call = pl.pallas_call(
    _head_body,
    out_shape=jax.ShapeDtypeStruct((G, D), jnp.float32),
)


def kernel(x, edge_index, batch, W1, b1, W2, b2, W3, b3, W4, b4):
    pad_e = NW * EPW - E
    srcp = jnp.concatenate(
        [edge_index[0], jnp.full((pad_e,), N, jnp.int32)]).reshape(NW, NCH, C)
    dstp = jnp.concatenate(
        [edge_index[1], jnp.full((pad_e,), N, jnp.int32)]).reshape(NW, NCH, C)
    xp = jnp.pad(x, ((0, NP - N), (0, 0)))
    z128 = jnp.zeros((RPT, D), jnp.float32)
    ones128 = jnp.ones((C, D), jnp.float32)

    degp = _deg_pass(dstp, ones128, z128)
    g1, dinv = _scale_call(degp, xp, W1)
    s1 = _edge_pass(g1, srcp, dstp, z128)
    g2 = _layer_call(s1, g1, dinv, b1.reshape(1, D), W2)
    s2 = _edge_pass(g2, srcp, dstp, z128)
    out = _head_call(s2, g2, dinv, b2.reshape(1, D), batch.reshape(1, N),
                     W3, b3.reshape(1, D), W4, b4.reshape(1, D))
    return out


# speedup vs baseline: 28.0549x; 2.3419x over previous
import functools

import jax
import jax.numpy as jnp
from jax import lax
from jax.experimental import pallas as pl
from jax.experimental.pallas import tpu as pltpu
from jax.experimental.pallas import tpu_sc as plsc

N = 10000
E = 320000
D = 128
G = 64
NC = 2
NS = 16
NW = NC * NS
C = 128
NCH = 80
EPW = NCH * C
RPT = 632
NP = NS * RPT
EPWR = E // NW


def _sc_mesh():
    return plsc.VectorSubcoreMesh(
        core_axis_name="c", subcore_axis_name="s", num_cores=NC, num_subcores=NS
    )


@functools.partial(
    pl.kernel,
    out_type=jax.ShapeDtypeStruct((NC, NP, D), jnp.float32),
    mesh=_sc_mesh(),
    scratch_types=[
        pltpu.VMEM((NCH, C), jnp.int32),
        pltpu.VMEM((C, D), jnp.float32),
        pltpu.VMEM_SHARED((NP, D), jnp.float32),
    ],
)
def _deg_pass(dst_hbm, ones_hbm, z_hbm, out_hbm, idx_v, ones_v, acc):
    c = lax.axis_index("c")
    s = lax.axis_index("s")
    wid = c * NS + s
    pltpu.sync_copy(z_hbm, acc.at[pl.ds(s * RPT, RPT)])
    pltpu.sync_copy(dst_hbm.at[wid], idx_v)
    pltpu.sync_copy(ones_hbm, ones_v)
    plsc.subcore_barrier()

    def body(j, carry):
        pltpu.sync_copy(ones_v, acc.at[idx_v.at[j]], add=True)
        return carry

    lax.fori_loop(0, NCH, body, 0)
    plsc.subcore_barrier()
    pltpu.sync_copy(acc.at[pl.ds(s * RPT, RPT)],
                    out_hbm.at[c, pl.ds(s * RPT, RPT)])


@functools.partial(
    pl.kernel,
    out_type=jax.ShapeDtypeStruct((NC, NP, D), jnp.float32),
    mesh=_sc_mesh(),
    scratch_types=[
        pltpu.VMEM((NCH // 2, C), jnp.int32),
        pltpu.VMEM((NCH // 2, C), jnp.int32),
        pltpu.VMEM((C, D), jnp.float32),
        pltpu.VMEM((C, D), jnp.float32),
        pltpu.VMEM_SHARED((NP, D), jnp.float32),
        pltpu.SemaphoreType.DMA,
        pltpu.SemaphoreType.DMA,
    ],
)
def _edge_pass(g_hbm, src_hbm, dst_hbm, z_hbm, out_hbm,
               isrc_v, idst_v, rows0, rows1, acc, sem0, sem1):
    c = lax.axis_index("c")
    s = lax.axis_index("s")
    wid = c * NS + s
    nchh = NCH // 2
    pltpu.sync_copy(z_hbm, acc.at[pl.ds(s * RPT, RPT)])
    plsc.subcore_barrier()

    for h in range(2):
        pltpu.sync_copy(src_hbm.at[wid, pl.ds(h * nchh, nchh)], isrc_v)
        pltpu.sync_copy(dst_hbm.at[wid, pl.ds(h * nchh, nchh)], idst_v)
        pltpu.async_copy(g_hbm.at[isrc_v.at[0]], rows0, sem0)
        pltpu.async_copy(g_hbm.at[isrc_v.at[1]], rows1, sem1)

        def body(i, carry):
            j = 2 * i
            pltpu.make_async_copy(g_hbm.at[isrc_v.at[j]], rows0, sem0).wait()
            pltpu.sync_copy(rows0, acc.at[idst_v.at[j]], add=True)

            @pl.when(j + 2 < nchh)
            def _():
                pltpu.async_copy(g_hbm.at[isrc_v.at[j + 2]], rows0, sem0)

            pltpu.make_async_copy(g_hbm.at[isrc_v.at[j + 1]], rows1, sem1).wait()
            pltpu.sync_copy(rows1, acc.at[idst_v.at[j + 1]], add=True)

            @pl.when(j + 3 < nchh)
            def _():
                pltpu.async_copy(g_hbm.at[isrc_v.at[j + 3]], rows1, sem1)

            return carry

        lax.fori_loop(0, nchh // 2, body, 0)

    plsc.subcore_barrier()
    pltpu.sync_copy(acc.at[pl.ds(s * RPT, RPT)],
                    out_hbm.at[c, pl.ds(s * RPT, RPT)])


def _scale_body(degp_ref, x_ref, w_ref, g_ref, dinv_ref):
    deg = 1.0 + degp_ref[0, :, 0:1] + degp_ref[1, :, 0:1]
    dinv = lax.rsqrt(deg)
    ht = jnp.dot(x_ref[...], w_ref[...], preferred_element_type=jnp.float32)
    g_ref[...] = dinv * ht
    dinv_ref[...] = dinv


_scale_call = pl.pallas_call(
    _scale_body,
    out_shape=(
        jax.ShapeDtypeStruct((NP, D), jnp.float32),
        jax.ShapeDtypeStruct((NP, 1), jnp.float32),
    ),
)


def _layer_body(s_ref, g_ref, dinv_ref, b_ref, w_ref, out_ref):
    agg = s_ref[0] + s_ref[1] + g_ref[...]
    h = jnp.maximum(dinv_ref[...] * agg + b_ref[...], 0.0)
    out_ref[...] = dinv_ref[...] * jnp.dot(
        h, w_ref[...], preferred_element_type=jnp.float32)


_layer_call = pl.pallas_call(
    _layer_body,
    out_shape=jax.ShapeDtypeStruct((NP, D), jnp.float32),
)


def _head_body(s_ref, g_ref, dinv_ref, b2_ref, batch_ref,
               w3_ref, b3_ref, w4_ref, b4_ref, out_ref):
    agg = s_ref[0] + s_ref[1] + g_ref[...]
    h = jnp.maximum(dinv_ref[...] * agg + b2_ref[...], 0.0)
    hr = h[:N]
    onehot_t = (lax.broadcasted_iota(jnp.int32, (G, N), 0)
                == batch_ref[...]).astype(jnp.float32)
    psum = jnp.dot(onehot_t, hr, preferred_element_type=jnp.float32)
    cnt = jnp.sum(onehot_t, axis=1, keepdims=True)
    pooled = psum / jnp.maximum(cnt, 1.0)
    z = jnp.maximum(
        jnp.dot(pooled, w3_ref[...], preferred_element_type=jnp.float32)
        + b3_ref[...], 0.0)
    out_ref[...] = jnp.dot(
        z, w4_ref[...], preferred_element_type=jnp.float32) + b4_ref[...]


_head_call = pl.pallas_call(
    _head_body,
    out_shape=jax.ShapeDtypeStruct((G, D), jnp.float32),
)


def kernel(x, edge_index, batch, W1, b1, W2, b2, W3, b3, W4, b4):
    pad_w = EPW - EPWR
    pad_idx = jnp.broadcast_to(
        N + (jnp.arange(pad_w, dtype=jnp.int32) % (NP - N)), (NW, pad_w))
    srcp = jnp.concatenate(
        [edge_index[0].reshape(NW, EPWR), pad_idx], axis=1).reshape(NW, NCH, C)
    dstp = jnp.concatenate(
        [edge_index[1].reshape(NW, EPWR), pad_idx], axis=1).reshape(NW, NCH, C)
    xp = jnp.pad(x, ((0, NP - N), (0, 0)))
    z128 = jnp.zeros((RPT, D), jnp.float32)
    ones128 = jnp.ones((C, D), jnp.float32)

    degp = _deg_pass(dstp, ones128, z128)
    g1, dinv = _scale_call(degp, xp, W1)
    s1 = _edge_pass(g1, srcp, dstp, z128)
    g2 = _layer_call(s1, g1, dinv, b1.reshape(1, D), W2)
    s2 = _edge_pass(g2, srcp, dstp, z128)
    out = _head_call(s2, g2, dinv, b2.reshape(1, D), batch.reshape(1, N),
                     W3, b3.reshape(1, D), W4, b4.reshape(1, D))
    return out


# speedup vs baseline: 28.2241x; 1.0060x over previous
import functools

import jax
import jax.numpy as jnp
from jax import lax
from jax.experimental import pallas as pl
from jax.experimental.pallas import tpu as pltpu
from jax.experimental.pallas import tpu_sc as plsc

N = 10000
E = 320000
D = 128
G = 64
NC = 2
NS = 16
NW = NC * NS
C = 128
NCH = 80
EPW = NCH * C
RPT = 640
NP = NS * RPT
EPWR = E // NW


def _sc_mesh():
    return plsc.VectorSubcoreMesh(
        core_axis_name="c", subcore_axis_name="s", num_cores=NC, num_subcores=NS
    )


DW = 128


@functools.partial(
    pl.kernel,
    out_type=jax.ShapeDtypeStruct((NC, NP, DW), jnp.float32),
    mesh=_sc_mesh(),
    scratch_types=[
        pltpu.VMEM((NCH, C), jnp.int32),
        pltpu.VMEM((C, DW), jnp.float32),
        pltpu.VMEM_SHARED((NP, DW), jnp.float32),
    ],
)
def _deg_pass(dst_hbm, ones_hbm, z_hbm, out_hbm, idx_v, ones_v, acc):
    c = lax.axis_index("c")
    s = lax.axis_index("s")
    wid = c * NS + s
    pltpu.sync_copy(z_hbm, acc.at[pl.ds(s * RPT, RPT)])
    pltpu.sync_copy(dst_hbm.at[wid], idx_v)
    pltpu.sync_copy(ones_hbm, ones_v)
    plsc.subcore_barrier()

    def body(j, carry):
        pltpu.sync_copy(ones_v, acc.at[idx_v.at[j]], add=True)
        return carry

    lax.fori_loop(0, NCH, body, 0)
    plsc.subcore_barrier()
    pltpu.sync_copy(acc.at[pl.ds(s * RPT, RPT)],
                    out_hbm.at[c, pl.ds(s * RPT, RPT)])


@functools.partial(
    pl.kernel,
    out_type=jax.ShapeDtypeStruct((NC, NP, D), jnp.float32),
    mesh=_sc_mesh(),
    scratch_types=[
        pltpu.VMEM((NCH // 2, C), jnp.int32),
        pltpu.VMEM((NCH // 2, C), jnp.int32),
        pltpu.VMEM((C, D), jnp.float32),
        pltpu.VMEM((C, D), jnp.float32),
        pltpu.VMEM_SHARED((NP, D), jnp.float32),
        pltpu.SemaphoreType.DMA,
        pltpu.SemaphoreType.DMA,
    ],
)
def _edge_pass(g_hbm, src_hbm, dst_hbm, z_hbm, out_hbm,
               isrc_v, idst_v, rows0, rows1, acc, sem0, sem1):
    c = lax.axis_index("c")
    s = lax.axis_index("s")
    wid = c * NS + s
    nchh = NCH // 2
    pltpu.sync_copy(z_hbm, acc.at[pl.ds(s * RPT, RPT)])
    plsc.subcore_barrier()

    for h in range(2):
        pltpu.sync_copy(src_hbm.at[wid, pl.ds(h * nchh, nchh)], isrc_v)
        pltpu.sync_copy(dst_hbm.at[wid, pl.ds(h * nchh, nchh)], idst_v)
        pltpu.async_copy(g_hbm.at[isrc_v.at[0]], rows0, sem0)
        pltpu.async_copy(g_hbm.at[isrc_v.at[1]], rows1, sem1)

        def body(i, carry):
            j = 2 * i
            pltpu.make_async_copy(g_hbm.at[isrc_v.at[j]], rows0, sem0).wait()
            pltpu.sync_copy(rows0, acc.at[idst_v.at[j]], add=True)

            @pl.when(j + 2 < nchh)
            def _():
                pltpu.async_copy(g_hbm.at[isrc_v.at[j + 2]], rows0, sem0)

            pltpu.make_async_copy(g_hbm.at[isrc_v.at[j + 1]], rows1, sem1).wait()
            pltpu.sync_copy(rows1, acc.at[idst_v.at[j + 1]], add=True)

            @pl.when(j + 3 < nchh)
            def _():
                pltpu.async_copy(g_hbm.at[isrc_v.at[j + 3]], rows1, sem1)

            return carry

        lax.fori_loop(0, nchh // 2, body, 0)

    plsc.subcore_barrier()
    pltpu.sync_copy(acc.at[pl.ds(s * RPT, RPT)],
                    out_hbm.at[c, pl.ds(s * RPT, RPT)])


def _scale_body(degp_ref, x_ref, w_ref, g_ref, dinv_ref):
    deg = 1.0 + degp_ref[0, :, 0:1] + degp_ref[1, :, 0:1]
    dinv = lax.rsqrt(deg)
    ht = jnp.dot(x_ref[...], w_ref[...], preferred_element_type=jnp.float32)
    g_ref[...] = dinv * ht
    dinv_ref[...] = dinv


_scale_call = pl.pallas_call(
    _scale_body,
    out_shape=(
        jax.ShapeDtypeStruct((NP, D), jnp.float32),
        jax.ShapeDtypeStruct((NP, 1), jnp.float32),
    ),
)


def _layer_body(s_ref, g_ref, dinv_ref, b_ref, w_ref, out_ref):
    agg = s_ref[0] + s_ref[1] + g_ref[...]
    h = jnp.maximum(dinv_ref[...] * agg + b_ref[...], 0.0)
    out_ref[...] = dinv_ref[...] * jnp.dot(
        h, w_ref[...], preferred_element_type=jnp.float32)


_layer_call = pl.pallas_call(
    _layer_body,
    out_shape=jax.ShapeDtypeStruct((NP, D), jnp.float32),
)


def _head_body(s_ref, g_ref, dinv_ref, b2_ref, batch_ref,
               w3_ref, b3_ref, w4_ref, b4_ref, out_ref):
    agg = s_ref[0] + s_ref[1] + g_ref[...]
    h = jnp.maximum(dinv_ref[...] * agg + b2_ref[...], 0.0)
    hr = h[:N]
    onehot_t = (lax.broadcasted_iota(jnp.int32, (G, N), 0)
                == batch_ref[...]).astype(jnp.float32)
    psum = jnp.dot(onehot_t, hr, preferred_element_type=jnp.float32)
    cnt = jnp.sum(onehot_t, axis=1, keepdims=True)
    pooled = psum / jnp.maximum(cnt, 1.0)
    z = jnp.maximum(
        jnp.dot(pooled, w3_ref[...], preferred_element_type=jnp.float32)
        + b3_ref[...], 0.0)
    out_ref[...] = jnp.dot(
        z, w4_ref[...], preferred_element_type=jnp.float32) + b4_ref[...]


_head_call = pl.pallas_call(
    _head_body,
    out_shape=jax.ShapeDtypeStruct((G, D), jnp.float32),
)


def kernel(x, edge_index, batch, W1, b1, W2, b2, W3, b3, W4, b4):
    pad_w = EPW - EPWR
    pad_idx = jnp.broadcast_to(
        N + (jnp.arange(pad_w, dtype=jnp.int32) % (NP - N)), (NW, pad_w))
    srcp = jnp.concatenate(
        [edge_index[0].reshape(NW, EPWR), pad_idx], axis=1).reshape(NW, NCH, C)
    dstp = jnp.concatenate(
        [edge_index[1].reshape(NW, EPWR), pad_idx], axis=1).reshape(NW, NCH, C)
    xp = jnp.pad(x, ((0, NP - N), (0, 0)))
    z128 = jnp.zeros((RPT, D), jnp.float32)
    zdw = jnp.zeros((RPT, DW), jnp.float32)
    onesdw = jnp.ones((C, DW), jnp.float32)

    degp = _deg_pass(dstp, onesdw, zdw)
    g1, dinv = _scale_call(degp, xp, W1)
    s1 = _edge_pass(g1, srcp, dstp, z128)
    g2 = _layer_call(s1, g1, dinv, b1.reshape(1, D), W2)
    s2 = _edge_pass(g2, srcp, dstp, z128)
    out = _head_call(s2, g2, dinv, b2.reshape(1, D), batch.reshape(1, N),
                     W3, b3.reshape(1, D), W4, b4.reshape(1, D))
    return out


# speedup vs baseline: 28.3279x; 1.0037x over previous
import functools

import jax
import jax.numpy as jnp
from jax import lax
from jax.experimental import pallas as pl
from jax.experimental.pallas import tpu as pltpu
from jax.experimental.pallas import tpu_sc as plsc

N = 10000
E = 320000
D = 128
G = 64
NC = 2
NS = 16
NW = NC * NS
C = 128
NCH = 80
EPW = NCH * C
RPT = 640
NP = NS * RPT
EPWR = E // NW


def _sc_mesh():
    return plsc.VectorSubcoreMesh(
        core_axis_name="c", subcore_axis_name="s", num_cores=NC, num_subcores=NS
    )


DW = 128


@functools.partial(
    pl.kernel,
    out_type=jax.ShapeDtypeStruct((NC, NP, DW), jnp.float32),
    mesh=_sc_mesh(),
    scratch_types=[
        pltpu.VMEM((NCH, C), jnp.int32),
        pltpu.VMEM((C, DW), jnp.float32),
        pltpu.VMEM_SHARED((NP, DW), jnp.float32),
    ],
)
def _deg_pass(dst_hbm, ones_hbm, z_hbm, out_hbm, idx_v, ones_v, acc):
    c = lax.axis_index("c")
    s = lax.axis_index("s")
    wid = c * NS + s
    pltpu.sync_copy(z_hbm, acc.at[pl.ds(s * RPT, RPT)])
    pltpu.sync_copy(dst_hbm.at[wid], idx_v)
    pltpu.sync_copy(ones_hbm, ones_v)
    plsc.subcore_barrier()

    def body(j, carry):
        pltpu.sync_copy(ones_v, acc.at[idx_v.at[j]], add=True)
        return carry

    lax.fori_loop(0, NCH, body, 0)
    plsc.subcore_barrier()
    pltpu.sync_copy(acc.at[pl.ds(s * RPT, RPT)],
                    out_hbm.at[c, pl.ds(s * RPT, RPT)])


@functools.partial(
    pl.kernel,
    out_type=jax.ShapeDtypeStruct((NC, NP, D), jnp.float32),
    mesh=_sc_mesh(),
    scratch_types=[
        pltpu.VMEM((NCH // 2, C), jnp.int32),
        pltpu.VMEM((NCH // 2, C), jnp.int32),
        pltpu.VMEM((C, D), jnp.float32),
        pltpu.VMEM((C, D), jnp.float32),
        pltpu.VMEM_SHARED((NP, D), jnp.float32),
        pltpu.SemaphoreType.DMA,
        pltpu.SemaphoreType.DMA,
    ],
)
def _edge_pass(g_hbm, src_hbm, dst_hbm, z_hbm, out_hbm,
               isrc_v, idst_v, rows0, rows1, acc, sem0, sem1):
    c = lax.axis_index("c")
    s = lax.axis_index("s")
    wid = c * NS + s
    nchh = NCH // 2
    pltpu.sync_copy(z_hbm, acc.at[pl.ds(s * RPT, RPT)])
    plsc.subcore_barrier()

    for h in range(2):
        pltpu.sync_copy(src_hbm.at[wid, pl.ds(h * nchh, nchh)], isrc_v)
        pltpu.sync_copy(dst_hbm.at[wid, pl.ds(h * nchh, nchh)], idst_v)
        pltpu.async_copy(g_hbm.at[isrc_v.at[0]], rows0, sem0)
        pltpu.async_copy(g_hbm.at[isrc_v.at[1]], rows1, sem1)

        def body(i, carry):
            j = 2 * i
            pltpu.make_async_copy(g_hbm.at[isrc_v.at[j]], rows0, sem0).wait()
            pltpu.sync_copy(rows0, acc.at[idst_v.at[j]], add=True)

            @pl.when(j + 2 < nchh)
            def _():
                pltpu.async_copy(g_hbm.at[isrc_v.at[j + 2]], rows0, sem0)

            pltpu.make_async_copy(g_hbm.at[isrc_v.at[j + 1]], rows1, sem1).wait()
            pltpu.sync_copy(rows1, acc.at[idst_v.at[j + 1]], add=True)

            @pl.when(j + 3 < nchh)
            def _():
                pltpu.async_copy(g_hbm.at[isrc_v.at[j + 3]], rows1, sem1)

            return carry

        lax.fori_loop(0, nchh // 2, body, 0)

    plsc.subcore_barrier()
    pltpu.sync_copy(acc.at[pl.ds(s * RPT, RPT)],
                    out_hbm.at[c, pl.ds(s * RPT, RPT)])


def _scale_body(degp_ref, x_ref, w_ref, g_ref, dinv_ref):
    deg = 1.0 + degp_ref[0, :, 0:1] + degp_ref[1, :, 0:1]
    dinv = lax.rsqrt(deg)
    ht = jnp.dot(x_ref[...], w_ref[...], preferred_element_type=jnp.float32)
    g_ref[pl.ds(0, N)] = dinv[0:N] * ht
    g_ref[pl.ds(N, NP - N)] = jnp.zeros((NP - N, D), jnp.float32)
    dinv_ref[...] = dinv


_scale_call = pl.pallas_call(
    _scale_body,
    out_shape=(
        jax.ShapeDtypeStruct((NP, D), jnp.float32),
        jax.ShapeDtypeStruct((NP, 1), jnp.float32),
    ),
)


def _layer_body(s_ref, g_ref, dinv_ref, b_ref, w_ref, out_ref):
    agg = s_ref[0] + s_ref[1] + g_ref[...]
    h = jnp.maximum(dinv_ref[...] * agg + b_ref[...], 0.0)
    out_ref[...] = dinv_ref[...] * jnp.dot(
        h, w_ref[...], preferred_element_type=jnp.float32)


_layer_call = pl.pallas_call(
    _layer_body,
    out_shape=jax.ShapeDtypeStruct((NP, D), jnp.float32),
)


def _head_body(s_ref, g_ref, dinv_ref, b2_ref, batch_ref,
               w3_ref, b3_ref, w4_ref, b4_ref, out_ref):
    agg = s_ref[0] + s_ref[1] + g_ref[...]
    h = jnp.maximum(dinv_ref[...] * agg + b2_ref[...], 0.0)
    hr = h[:N]
    onehot_t = (lax.broadcasted_iota(jnp.int32, (G, N), 0)
                == batch_ref[...]).astype(jnp.float32)
    psum = jnp.dot(onehot_t, hr, preferred_element_type=jnp.float32)
    cnt = jnp.sum(onehot_t, axis=1, keepdims=True)
    pooled = psum / jnp.maximum(cnt, 1.0)
    z = jnp.maximum(
        jnp.dot(pooled, w3_ref[...], preferred_element_type=jnp.float32)
        + b3_ref[...], 0.0)
    out_ref[...] = jnp.dot(
        z, w4_ref[...], preferred_element_type=jnp.float32) + b4_ref[...]


_head_call = pl.pallas_call(
    _head_body,
    out_shape=jax.ShapeDtypeStruct((G, D), jnp.float32),
)


def kernel(x, edge_index, batch, W1, b1, W2, b2, W3, b3, W4, b4):
    pad_w = EPW - EPWR
    pad_idx = jnp.broadcast_to(
        N + (jnp.arange(pad_w, dtype=jnp.int32) % (NP - N)), (NW, pad_w))
    srcp = jnp.concatenate(
        [edge_index[0].reshape(NW, EPWR), pad_idx], axis=1).reshape(NW, NCH, C)
    dstp = jnp.concatenate(
        [edge_index[1].reshape(NW, EPWR), pad_idx], axis=1).reshape(NW, NCH, C)
    z128 = jnp.zeros((RPT, D), jnp.float32)
    zdw = jnp.zeros((RPT, DW), jnp.float32)
    onesdw = jnp.ones((C, DW), jnp.float32)

    degp = _deg_pass(dstp, onesdw, zdw)
    g1, dinv = _scale_call(degp, x, W1)
    s1 = _edge_pass(g1, srcp, dstp, z128)
    g2 = _layer_call(s1, g1, dinv, b1.reshape(1, D), W2)
    s2 = _edge_pass(g2, srcp, dstp, z128)
    out = _head_call(s2, g2, dinv, b2.reshape(1, D), batch.reshape(1, N),
                     W3, b3.reshape(1, D), W4, b4.reshape(1, D))
    return out
